# baseline (speedup 1.0000x reference)
import functools

import jax
import jax.numpy as jnp
from jax import lax
from jax.experimental import pallas as pl
from jax.experimental.pallas import tpu as pltpu
from jax.experimental.pallas import tpu_sc as plsc

D_MODEL_K = 1024
SCALE = 32.0
B_TOTAL = 16384
SEQ = 4096
NC = 2
NS = 16
NW = NC * NS
RPW = B_TOTAL // NW
CHUNK = 16
NCHUNK = RPW // CHUNK
LANES = 16
COLS = D_MODEL_K // LANES


def _make_kernel():
    mesh = plsc.VectorSubcoreMesh(core_axis_name="c", subcore_axis_name="s")

    @functools.partial(
        pl.kernel,
        mesh=mesh,
        out_type=jax.ShapeDtypeStruct((B_TOTAL, D_MODEL_K), jnp.float32),
        scratch_types=[
            pltpu.VMEM((NCHUNK, CHUNK), jnp.int32),
            pltpu.VMEM((CHUNK, D_MODEL_K), jnp.float32),
            pltpu.VMEM((CHUNK, D_MODEL_K), jnp.float32),
            pltpu.VMEM((CHUNK, D_MODEL_K), jnp.float32),
            pltpu.VMEM((CHUNK, D_MODEL_K), jnp.float32),
            pltpu.SemaphoreType.DMA,
            pltpu.SemaphoreType.DMA,
            pltpu.SemaphoreType.DMA,
            pltpu.SemaphoreType.DMA,
            pltpu.SemaphoreType.DMA,
            pltpu.SemaphoreType.DMA,
        ],
    )
    def k(x_hbm, table_hbm, enc_hbm, out_hbm, idx_v, rows0, rows1, enc0, enc1,
          gsem0, gsem1, esem0, esem1, ssem0, ssem1):
        cid = lax.axis_index("c")
        sid = lax.axis_index("s")
        wid = sid * NC + cid
        base = wid * RPW
        seq_base = lax.rem(base, SEQ)

        rows = (rows0, rows1)
        enc = (enc0, enc1)
        gsem = (gsem0, gsem1)
        esem = (esem0, esem1)
        ssem = (ssem0, ssem1)

        pltpu.sync_copy(x_hbm.at[pl.ds(wid * NCHUNK, NCHUNK)], idx_v)

        def issue(g, p):
            pltpu.async_copy(table_hbm.at[idx_v.at[g]], rows[p], gsem[p])
            soff = seq_base + g * CHUNK
            pltpu.async_copy(enc_hbm.at[pl.ds(soff, CHUNK)], enc[p], esem[p])

        def wait_in(p):
            pltpu.make_async_copy(
                table_hbm.at[pl.ds(0, CHUNK)], rows[p], gsem[p]).wait()
            pltpu.make_async_copy(
                enc_hbm.at[pl.ds(0, CHUNK)], enc[p], esem[p]).wait()

        def issue_store(g, p):
            off = base + g * CHUNK
            pltpu.async_copy(rows[p], out_hbm.at[pl.ds(off, CHUNK)], ssem[p])

        def wait_store(p):
            pltpu.make_async_copy(
                rows[p], out_hbm.at[pl.ds(0, CHUNK)], ssem[p]).wait()

        def compute(p):
            def row_body(i, _):
                def col_body(j, _):
                    sl = pl.ds(pl.multiple_of(j * LANES, LANES), LANES)
                    rows[p][i, sl] = rows[p][i, sl] * SCALE + enc[p][i, sl]
                    return 0

                lax.fori_loop(0, COLS, col_body, 0, unroll=8)
                return 0

            lax.fori_loop(0, CHUNK, row_body, 0)

        issue(0, 0)

        def step(t, _):
            g0 = t * 2
            @pl.when(t > 0)
            def _():
                wait_store(1)
            issue(g0 + 1, 1)
            wait_in(0)
            compute(0)
            issue_store(g0, 0)
            wait_store(0)
            @pl.when(t < NCHUNK // 2 - 1)
            def _():
                issue(g0 + 2, 0)
            wait_in(1)
            compute(1)
            issue_store(g0 + 1, 1)
            return 0

        lax.fori_loop(0, NCHUNK // 2, step, 0)
        wait_store(1)

    return k


_sc_embed = _make_kernel()


def kernel(x, table, encoding):
    x_idx = x.reshape(NW * NCHUNK, CHUNK).astype(jnp.int32)
    out = _sc_embed(x_idx, table, encoding)
    return out.reshape(x.shape[0], x.shape[1], D_MODEL_K)


# speedup vs baseline: 1.8504x; 1.8504x over previous
import functools

import jax
import jax.numpy as jnp
from jax import lax
from jax.experimental import pallas as pl
from jax.experimental.pallas import tpu as pltpu
from jax.experimental.pallas import tpu_sc as plsc

D_MODEL_K = 1024
SCALE = 32.0
BATCH = 4
SEQ = 4096
NC = 2
NS = 16
NW = NC * NS
SPW = SEQ // NW
CHUNK = 8
NT = SPW // CHUNK
LANES = 16
COLS = D_MODEL_K // LANES
XROWS = BATCH * SEQ // CHUNK


def _make_kernel():
    mesh = plsc.VectorSubcoreMesh(core_axis_name="c", subcore_axis_name="s")

    @functools.partial(
        pl.kernel,
        mesh=mesh,
        out_type=jax.ShapeDtypeStruct((BATCH * SEQ, D_MODEL_K), jnp.float32),
        scratch_types=[
            pltpu.VMEM((BATCH, NT, CHUNK), jnp.int32),
            pltpu.VMEM((BATCH, CHUNK, D_MODEL_K), jnp.float32),
            pltpu.VMEM((8, CHUNK, D_MODEL_K), jnp.float32),
            pltpu.VMEM((2, CHUNK, D_MODEL_K), jnp.float32),
            pltpu.SemaphoreType.DMA((BATCH,)),
            pltpu.SemaphoreType.DMA((8,)),
            pltpu.SemaphoreType.DMA((2,)),
        ],
    )
    def k(x_hbm, table_hbm, enc_hbm, out_hbm, idx_v, rows_v, ob_v, enc_v,
          gsem, ssem, esem):
        cid = lax.axis_index("c")
        sid = lax.axis_index("s")
        wid = sid * NC + cid
        seq0 = wid * SPW

        for b in range(BATCH):
            pltpu.sync_copy(
                x_hbm.at[pl.ds(b * (SEQ // CHUNK) + wid * NT, NT)],
                idx_v.at[b])

        def obuf(par, b):
            return 4 * par + b

        def issue_gather(t, b):
            pltpu.async_copy(table_hbm.at[idx_v.at[b, t]],
                             rows_v.at[b], gsem.at[b])

        def wait_gather(b):
            pltpu.make_async_copy(
                table_hbm.at[pl.ds(0, CHUNK)], rows_v.at[b],
                gsem.at[b]).wait()

        def issue_enc(t, e):
            pltpu.async_copy(enc_hbm.at[pl.ds(seq0 + t * CHUNK, CHUNK)],
                             enc_v.at[e], esem.at[e])

        def wait_enc(e):
            pltpu.make_async_copy(
                enc_hbm.at[pl.ds(0, CHUNK)], enc_v.at[e], esem.at[e]).wait()

        def issue_store(t, par, b):
            off = b * SEQ + seq0 + t * CHUNK
            pltpu.async_copy(ob_v.at[obuf(par, b)],
                             out_hbm.at[pl.ds(off, CHUNK)],
                             ssem.at[obuf(par, b)])

        def wait_store(par, b):
            pltpu.make_async_copy(
                ob_v.at[obuf(par, b)], out_hbm.at[pl.ds(0, CHUNK)],
                ssem.at[obuf(par, b)]).wait()

        def compute(par, b, e):
            o = obuf(par, b)

            def row_body(i, _):
                def col_body(j, _):
                    sl = pl.ds(pl.multiple_of(j * LANES, LANES), LANES)
                    ob_v[o, i, sl] = (rows_v[b, i, sl] * SCALE
                                      + enc_v[e, i, sl])
                    return 0

                lax.fori_loop(0, COLS, col_body, 0, unroll=16)
                return 0

            lax.fori_loop(0, CHUNK, row_body, 0)

        issue_enc(0, 0)
        issue_enc(1, 1)
        for b in range(BATCH):
            issue_gather(0, b)

        def step(tt, _):
            for par in (0, 1):
                t = tt * 2 + par
                e = par
                wait_enc(e)
                for b in range(BATCH):
                    wait_gather(b)
                    @pl.when(t >= 2)
                    def _():
                        wait_store(par, b)
                    compute(par, b, e)
                    issue_store(t, par, b)
                    @pl.when(t + 1 < NT)
                    def _():
                        issue_gather(t + 1, b)
                @pl.when(t + 2 < NT)
                def _():
                    issue_enc(t + 2, e)
            return 0

        lax.fori_loop(0, NT // 2, step, 0)
        for par in (0, 1):
            for b in range(BATCH):
                wait_store(par, b)

    return k


_sc_embed = _make_kernel()


def kernel(x, table, encoding):
    x_idx = x.reshape(XROWS, CHUNK).astype(jnp.int32)
    out = _sc_embed(x_idx, table, encoding)
    return out.reshape(x.shape[0], x.shape[1], D_MODEL_K)


# speedup vs baseline: 1.8554x; 1.0027x over previous
import functools

import jax
import jax.numpy as jnp
from jax import lax
from jax.experimental import pallas as pl
from jax.experimental.pallas import tpu as pltpu
from jax.experimental.pallas import tpu_sc as plsc

D_MODEL_K = 1024
SCALE = 32.0
BATCH = 4
SEQ = 4096
NC = 2
NS = 16
NW = NC * NS
SPW = SEQ // NW
CHUNK = 8
NT = SPW // CHUNK
LANES = 16
COLS = D_MODEL_K // LANES
XROWS = BATCH * SEQ // CHUNK


def _make_kernel():
    mesh = plsc.VectorSubcoreMesh(core_axis_name="c", subcore_axis_name="s")

    @functools.partial(
        pl.kernel,
        mesh=mesh,
        out_type=jax.ShapeDtypeStruct((BATCH * SEQ, D_MODEL_K), jnp.float32),
        scratch_types=[
            pltpu.VMEM((BATCH, NT, CHUNK), jnp.int32),
            pltpu.VMEM((BATCH, CHUNK, D_MODEL_K), jnp.float32),
            pltpu.VMEM((8, CHUNK, D_MODEL_K), jnp.float32),
            pltpu.VMEM((2, CHUNK, D_MODEL_K), jnp.float32),
            pltpu.SemaphoreType.DMA((BATCH,)),
            pltpu.SemaphoreType.DMA((8,)),
            pltpu.SemaphoreType.DMA((2,)),
        ],
    )
    def k(x_hbm, table_hbm, enc_hbm, out_hbm, idx_v, rows_v, ob_v, enc_v,
          gsem, ssem, esem):
        cid = lax.axis_index("c")
        sid = lax.axis_index("s")
        wid = sid * NC + cid
        seq0 = wid * SPW

        for b in range(BATCH):
            pltpu.sync_copy(
                x_hbm.at[pl.ds(b * (SEQ // CHUNK) + wid * NT, NT)],
                idx_v.at[b])

        def obuf(par, b):
            return 4 * par + b

        def issue_gather(t, b):
            pltpu.async_copy(table_hbm.at[idx_v.at[b, t]],
                             rows_v.at[b], gsem.at[b])

        def wait_gather(b):
            pltpu.make_async_copy(
                table_hbm.at[pl.ds(0, CHUNK)], rows_v.at[b],
                gsem.at[b]).wait()

        def issue_enc(t, e):
            pltpu.async_copy(enc_hbm.at[pl.ds(seq0 + t * CHUNK, CHUNK)],
                             enc_v.at[e], esem.at[e])

        def wait_enc(e):
            pltpu.make_async_copy(
                enc_hbm.at[pl.ds(0, CHUNK)], enc_v.at[e], esem.at[e]).wait()

        def issue_store(t, par, b):
            off = b * SEQ + seq0 + t * CHUNK
            pltpu.async_copy(ob_v.at[obuf(par, b)],
                             out_hbm.at[pl.ds(off, CHUNK)],
                             ssem.at[obuf(par, b)])

        def wait_store(par, b):
            pltpu.make_async_copy(
                ob_v.at[obuf(par, b)], out_hbm.at[pl.ds(0, CHUNK)],
                ssem.at[obuf(par, b)]).wait()

        def compute(par, b, e):
            o = obuf(par, b)

            def row_body(i, _):
                def col_body(j, _):
                    sl = pl.ds(pl.multiple_of(j * LANES, LANES), LANES)
                    ob_v[o, i, sl] = (rows_v[b, i, sl] * SCALE
                                      + enc_v[e, i, sl])
                    return 0

                lax.fori_loop(0, COLS, col_body, 0, unroll=8)
                return 0

            lax.fori_loop(0, CHUNK, row_body, 0)

        issue_enc(0, 0)
        issue_enc(1, 1)
        for b in range(BATCH):
            issue_gather(0, b)

        def step(tt, _):
            for par in (0, 1):
                t = tt * 2 + par
                e = par
                wait_enc(e)
                for b in range(BATCH):
                    wait_gather(b)
                    @pl.when(t >= 2)
                    def _():
                        wait_store(par, b)
                    compute(par, b, e)
                    issue_store(t, par, b)
                    @pl.when(t + 1 < NT)
                    def _():
                        issue_gather(t + 1, b)
                @pl.when(t + 2 < NT)
                def _():
                    issue_enc(t + 2, e)
            return 0

        lax.fori_loop(0, NT // 2, step, 0)
        for par in (0, 1):
            for b in range(BATCH):
                wait_store(par, b)

    return k


_sc_embed = _make_kernel()


def kernel(x, table, encoding):
    x_idx = x.reshape(XROWS, CHUNK).astype(jnp.int32)
    out = _sc_embed(x_idx, table, encoding)
    return out.reshape(x.shape[0], x.shape[1], D_MODEL_K)
